# Initial kernel scaffold; baseline (speedup 1.0000x reference)
import jax
import jax.numpy as jnp
from jax.experimental import pallas as pl


def kernel(question_seq, correct_seq, k_emb, v_emb, Mk, Mv0, fW, fb, eW, eb, aW, ab, pW, pb):
    raise NotImplementedError("write your pallas kernel here")


# speedup vs baseline: 2.4589x; 2.4589x over previous
import jax
import jax.numpy as jnp
from jax.experimental import pallas as pl
from jax.experimental.pallas import tpu as pltpu
from functools import partial

BB = 16


def _dkvmn_kernel(num_q, S,
                  qs_ref, cs_ref,
                  kemb_ref, vemb_ref,
                  mkT_ref, mv0_ref,
                  fW1_ref, fW2_ref, eW_ref, aW_ref,
                  fb_ref, eb_ref, ab_ref, pWt_ref, pb_ref,
                  out_ref,
                  kbuf, vbuf, wbuf, ebuf, abuf, rbuf, mv_scr):
    i = pl.program_id(0)
    gb = i * BB

    def gather_body(s, _):
        off = pl.multiple_of(s * BB, BB)
        for b in range(BB):
            q = qs_ref[gb + b, s]
            c = cs_ref[gb + b, s]
            kbuf[off + b] = kemb_ref[q, 0]
            vbuf[off + b] = vemb_ref[q + c * num_q, 0]
        return 0

    jax.lax.fori_loop(0, S, gather_body, 0)

    kmat = kbuf[...]
    vmat = vbuf[...]
    logits = jnp.dot(kmat, mkT_ref[...], preferred_element_type=jnp.float32)
    wbuf[...] = jax.nn.softmax(logits, axis=-1)
    ebuf[...] = jax.nn.sigmoid(
        jnp.dot(vmat, eW_ref[...], preferred_element_type=jnp.float32)
        + eb_ref[...])
    abuf[...] = jnp.tanh(
        jnp.dot(vmat, aW_ref[...], preferred_element_type=jnp.float32)
        + ab_ref[...])

    mv_scr[...] = jnp.broadcast_to(mv0_ref[...][None], mv_scr.shape)

    def scan_body(s, _):
        off = pl.multiple_of(s * BB, BB)
        w_t = wbuf[pl.ds(off, BB), :]
        e_t = ebuf[pl.ds(off, BB), :]
        a_t = abuf[pl.ds(off, BB), :]
        M = mv_scr[...]
        w3 = w_t[:, :, None]
        rbuf[pl.ds(off, BB), :] = jnp.sum(M * w3, axis=1)
        mv_scr[...] = M * (1.0 - w3 * e_t[:, None, :]) + w3 * a_t[:, None, :]
        return 0

    jax.lax.fori_loop(0, S, scan_body, 0)

    f = jnp.tanh(
        jnp.dot(rbuf[...], fW1_ref[...], preferred_element_type=jnp.float32)
        + jnp.dot(kmat, fW2_ref[...], preferred_element_type=jnp.float32)
        + fb_ref[...])
    f3 = f.reshape(S, BB, f.shape[-1])
    logit = jnp.sum(f3 * pWt_ref[...][None], axis=-1) + pb_ref[...]
    out_ref[...] = jax.nn.sigmoid(logit)[None]


def kernel(question_seq, correct_seq, k_emb, v_emb, Mk, Mv0, fW, fb, eW, eb,
           aW, ab, pW, pb):
    B, S = question_seq.shape
    num_q, DK = k_emb.shape
    DV = Mk.shape[0]
    nb = B // BB

    qs = question_seq.astype(jnp.int32)
    cs = correct_seq.astype(jnp.int32)
    kemb3 = k_emb.reshape(num_q, 1, DK)
    vemb3 = v_emb.reshape(2 * num_q, 1, DK)
    mkT = Mk.T
    fW1 = fW[:DK]
    fW2 = fW[DK:]
    fb2 = fb.reshape(1, DK)
    eb2 = eb.reshape(1, DK)
    ab2 = ab.reshape(1, DK)
    pWt = pW.reshape(1, DK)
    pb2 = pb.reshape(1, 1)

    M = S * BB
    out = pl.pallas_call(
        partial(_dkvmn_kernel, num_q, S),
        out_shape=jax.ShapeDtypeStruct((nb, S, BB), jnp.float32),
        grid=(nb,),
        in_specs=[
            pl.BlockSpec(memory_space=pltpu.SMEM),
            pl.BlockSpec(memory_space=pltpu.SMEM),
        ] + [pl.BlockSpec(memory_space=pltpu.VMEM)] * 13,
        out_specs=pl.BlockSpec((1, S, BB), lambda i: (i, 0, 0)),
        scratch_shapes=[
            pltpu.VMEM((M, DK), jnp.float32),
            pltpu.VMEM((M, DK), jnp.float32),
            pltpu.VMEM((M, DV), jnp.float32),
            pltpu.VMEM((M, DK), jnp.float32),
            pltpu.VMEM((M, DK), jnp.float32),
            pltpu.VMEM((M, DK), jnp.float32),
            pltpu.VMEM((BB, DV, DK), jnp.float32),
        ],
        compiler_params=pltpu.CompilerParams(
            dimension_semantics=("parallel",),
            vmem_limit_bytes=56 * 1024 * 1024,
        ),
        name="dkvmn_fused",
    )(qs, cs, kemb3, vemb3, mkT, Mv0, fW1, fW2, eW, aW, fb2, eb2, ab2, pWt,
      pb2)
    return out.transpose(0, 2, 1).reshape(B, S)


# speedup vs baseline: 2.9009x; 1.1798x over previous
import jax
import jax.numpy as jnp
from jax.experimental import pallas as pl
from jax.experimental.pallas import tpu as pltpu
from functools import partial

BB = 32


def _dkvmn_kernel(num_q, S,
                  qs_ref, cs_ref,
                  kemb_ref, vemb_ref,
                  mkT_ref, mv0_ref,
                  fW1_ref, fW2_ref, eW_ref, aW_ref,
                  fb_ref, eb_ref, ab_ref, pWt_ref, pb_ref,
                  out_ref,
                  kbuf, vbuf, wbuf, ebuf, abuf, rbuf, mv_scr):
    i = pl.program_id(0)
    gb = i * BB

    def gather_body(s, _):
        off = pl.multiple_of(s * BB, BB)
        for b in range(BB):
            q = qs_ref[gb + b, s]
            c = cs_ref[gb + b, s]
            kbuf[off + b] = kemb_ref[q, 0]
            vbuf[off + b] = vemb_ref[q + c * num_q, 0]
        return 0

    jax.lax.fori_loop(0, S, gather_body, 0)

    kmat = kbuf[...]
    vmat = vbuf[...]
    logits = jnp.dot(kmat, mkT_ref[...], preferred_element_type=jnp.float32)
    wbuf[...] = jax.nn.softmax(logits, axis=-1)
    ebuf[...] = jax.nn.sigmoid(
        jnp.dot(vmat, eW_ref[...], preferred_element_type=jnp.float32)
        + eb_ref[...])
    abuf[...] = jnp.tanh(
        jnp.dot(vmat, aW_ref[...], preferred_element_type=jnp.float32)
        + ab_ref[...])

    mv_scr[...] = jnp.broadcast_to(mv0_ref[...][None], mv_scr.shape)

    def scan_body(s, _):
        off = pl.multiple_of(s * BB, BB)
        w_t = wbuf[pl.ds(off, BB), :]
        e_t = ebuf[pl.ds(off, BB), :]
        a_t = abuf[pl.ds(off, BB), :]
        M = mv_scr[...]
        w3 = w_t[:, :, None]
        P = M * w3
        rbuf[pl.ds(off, BB), :] = jnp.sum(P, axis=1)
        mv_scr[...] = M - P * e_t[:, None, :] + w3 * a_t[:, None, :]
        return 0

    jax.lax.fori_loop(0, S, scan_body, 0)

    f = jnp.tanh(
        jnp.dot(rbuf[...], fW1_ref[...], preferred_element_type=jnp.float32)
        + jnp.dot(kmat, fW2_ref[...], preferred_element_type=jnp.float32)
        + fb_ref[...])
    f3 = f.reshape(S, BB, f.shape[-1])
    logit = jnp.sum(f3 * pWt_ref[...][None], axis=-1) + pb_ref[...]
    out_ref[...] = jax.nn.sigmoid(logit)[None]


def kernel(question_seq, correct_seq, k_emb, v_emb, Mk, Mv0, fW, fb, eW, eb,
           aW, ab, pW, pb):
    B, S = question_seq.shape
    num_q, DK = k_emb.shape
    DV = Mk.shape[0]
    nb = B // BB

    qs = question_seq.astype(jnp.int32)
    cs = correct_seq.astype(jnp.int32)
    kemb3 = k_emb.reshape(num_q, 1, DK)
    vemb3 = v_emb.reshape(2 * num_q, 1, DK)
    mkT = Mk.T
    fW1 = fW[:DK]
    fW2 = fW[DK:]
    fb2 = fb.reshape(1, DK)
    eb2 = eb.reshape(1, DK)
    ab2 = ab.reshape(1, DK)
    pWt = pW.reshape(1, DK)
    pb2 = pb.reshape(1, 1)

    M = S * BB
    out = pl.pallas_call(
        partial(_dkvmn_kernel, num_q, S),
        out_shape=jax.ShapeDtypeStruct((nb, S, BB), jnp.float32),
        grid=(nb,),
        in_specs=[
            pl.BlockSpec(memory_space=pltpu.SMEM),
            pl.BlockSpec(memory_space=pltpu.SMEM),
        ] + [pl.BlockSpec(memory_space=pltpu.VMEM)] * 13,
        out_specs=pl.BlockSpec((1, S, BB), lambda i: (i, 0, 0)),
        scratch_shapes=[
            pltpu.VMEM((M, DK), jnp.float32),
            pltpu.VMEM((M, DK), jnp.float32),
            pltpu.VMEM((M, DV), jnp.float32),
            pltpu.VMEM((M, DK), jnp.float32),
            pltpu.VMEM((M, DK), jnp.float32),
            pltpu.VMEM((M, DK), jnp.float32),
            pltpu.VMEM((BB, DV, DK), jnp.float32),
        ],
        compiler_params=pltpu.CompilerParams(
            dimension_semantics=("parallel",),
            vmem_limit_bytes=56 * 1024 * 1024,
        ),
        name="dkvmn_fused",
    )(qs, cs, kemb3, vemb3, mkT, Mv0, fW1, fW2, eW, aW, fb2, eb2, ab2, pWt,
      pb2)
    return out.transpose(0, 2, 1).reshape(B, S)
